# baseline (speedup 1.0000x reference)
import functools

import jax
import jax.numpy as jnp
from jax import lax
from jax.experimental import pallas as pl
from jax.experimental.pallas import tpu as pltpu
from jax.experimental.pallas import tpu_sc as plsc

N = 10000
NFEAT = 128
NCLASS = 128
E = 320000

NC = 2
NS = 16
NW = NC * NS
CH = 128
EPW = -(-E // (NW * CH)) * CH
NCHUNK = EPW // CH
K0 = 60
K1 = 2 * NCHUNK - K0
N_PAD = 10240
ROWS_PER_TILE = N_PAD // NS
LANES = 16
FV = NCLASS // LANES


def _linear_body(x_ref, w_ref, b_ref, o_ref):
    o_ref[...] = (
        jnp.dot(x_ref[...], w_ref[...], preferred_element_type=jnp.float32)
        + b_ref[...]
    )


def _linear(x, W, b):
    blk = N // 10
    return pl.pallas_call(
        _linear_body,
        grid=(N // blk,),
        in_specs=[
            pl.BlockSpec((blk, NFEAT), lambda i: (i, 0)),
            pl.BlockSpec((NFEAT, NCLASS), lambda i: (0, 0)),
            pl.BlockSpec((1, NCLASS), lambda i: (0, 0)),
        ],
        out_specs=pl.BlockSpec((blk, NCLASS), lambda i: (i, 0)),
        out_shape=jax.ShapeDtypeStruct((N, NCLASS), jnp.float32),
    )(x, W, b.reshape(1, NCLASS))


def _spmm_body(h_hbm, epack_hbm, zeros_hbm, out_hbm,
               acc, ebuf, gbuf, gsem):
    cid = lax.axis_index("c")
    sid = lax.axis_index("s")
    w = sid * NC + cid

    pltpu.sync_copy(zeros_hbm,
                    acc.at[pl.ds(sid * ROWS_PER_TILE, ROWS_PER_TILE)])
    plsc.subcore_barrier()

    nk = jnp.where(cid == 0, K0, K1)

    def chunk(k, carry):
        pltpu.sync_copy(epack_hbm.at[w, k], ebuf)
        pltpu.async_copy(h_hbm.at[ebuf.at[1]], gbuf, gsem).wait()
        for g in range(CH // LANES):
            vgroup = lax.bitcast_convert_type(
                ebuf[2, pl.ds(g * LANES, LANES)], jnp.float32)
            for j in range(LANES):
                e = g * LANES + j
                bvec = jnp.full((LANES,), vgroup[j], dtype=jnp.float32)
                for f in range(FV):
                    sl = pl.ds(f * LANES, LANES)
                    gbuf[e, sl] = gbuf[e, sl] * bvec
        pltpu.sync_copy(gbuf, acc.at[ebuf.at[0]], add=True)
        return carry

    lax.fori_loop(0, nk, chunk, 0)

    plsc.subcore_barrier()
    pltpu.sync_copy(acc.at[pl.ds(sid * ROWS_PER_TILE, ROWS_PER_TILE)],
                    out_hbm.at[cid, pl.ds(sid * ROWS_PER_TILE, ROWS_PER_TILE)])


_spmm = functools.partial(
    pl.kernel,
    out_type=jax.ShapeDtypeStruct((NC, N_PAD, NCLASS), jnp.float32),
    mesh=plsc.VectorSubcoreMesh(core_axis_name="c", subcore_axis_name="s",
                                num_cores=NC, num_subcores=NS),
    scratch_types=[
        pltpu.VMEM_SHARED((N_PAD, NCLASS), jnp.float32),
        pltpu.VMEM((3, CH), jnp.int32),
        pltpu.VMEM((CH, NCLASS), jnp.float32),
        pltpu.SemaphoreType.DMA,
    ],
)(_spmm_body)


def _combine_body(p_ref, o_ref):
    o_ref[...] = p_ref[0] + p_ref[1]


def _combine(partials):
    blk = 2000
    return pl.pallas_call(
        _combine_body,
        grid=(N // blk,),
        in_specs=[pl.BlockSpec((NC, blk, NCLASS), lambda i: (0, i, 0))],
        out_specs=pl.BlockSpec((blk, NCLASS), lambda i: (i, 0)),
        out_shape=jax.ShapeDtypeStruct((N, NCLASS), jnp.float32),
    )(partials)


def _skew(a):
    a = a.reshape(NS, 2 * NCHUNK, CH)
    c0 = jnp.pad(a[:, :K0], ((0, 0), (0, K1 - K0), (0, 0)))
    c1 = a[:, K0:]
    return jnp.stack([c0, c1], axis=1).reshape(NW, K1, CH)


def kernel(x, S_indices, S_values, W, b):
    h = _linear(x, W, b)
    e_pad = NW * EPW - E
    rows = jnp.pad(S_indices[0], (0, e_pad))
    cols = jnp.pad(S_indices[1], (0, e_pad))
    vals = jnp.pad(S_values, (0, e_pad)).view(jnp.int32)
    epack = jnp.stack([_skew(rows), _skew(cols), _skew(vals)], axis=2)
    zeros = jnp.zeros((ROWS_PER_TILE, NCLASS), jnp.float32)
    partials = _spmm(h, epack, zeros)
    return _combine(partials)


# speedup vs baseline: 1.3024x; 1.3024x over previous
import functools

import jax
import jax.numpy as jnp
from jax import lax
from jax.experimental import pallas as pl
from jax.experimental.pallas import tpu as pltpu
from jax.experimental.pallas import tpu_sc as plsc

N = 10000
NFEAT = 128
NCLASS = 128
E = 320000

NC = 2
NS = 16
NW = NC * NS
CH = 128
EPW = -(-E // (NW * CH)) * CH
NCHUNK = EPW // CH
K0 = 100
K1 = 2 * NCHUNK - K0
KMAX = max(K0, K1)
N_PAD = 10240
ROWS_PER_TILE = N_PAD // NS
LANES = 16
FV = NCLASS // LANES


def _linear_body(x_ref, w_ref, b_ref, o_ref):
    o_ref[...] = (
        jnp.dot(x_ref[...], w_ref[...], preferred_element_type=jnp.float32)
        + b_ref[...]
    )


def _linear(x, W, b):
    blk = N // 10
    return pl.pallas_call(
        _linear_body,
        grid=(N // blk,),
        in_specs=[
            pl.BlockSpec((blk, NFEAT), lambda i: (i, 0)),
            pl.BlockSpec((NFEAT, NCLASS), lambda i: (0, 0)),
            pl.BlockSpec((1, NCLASS), lambda i: (0, 0)),
        ],
        out_specs=pl.BlockSpec((blk, NCLASS), lambda i: (i, 0)),
        out_shape=jax.ShapeDtypeStruct((N, NCLASS), jnp.float32),
    )(x, W, b.reshape(1, NCLASS))


def _spmm_body(h_hbm, epack_hbm, zeros_hbm, out_hbm,
               acc, ebuf, gbuf, gsem):
    cid = lax.axis_index("c")
    sid = lax.axis_index("s")
    w = sid * NC + cid

    pltpu.sync_copy(zeros_hbm,
                    acc.at[pl.ds(sid * ROWS_PER_TILE, ROWS_PER_TILE)])
    plsc.subcore_barrier()

    nk = jnp.where(cid == 0, K0, K1)

    def chunk(k, carry):
        pltpu.sync_copy(epack_hbm.at[w, k], ebuf)
        pltpu.async_copy(h_hbm.at[ebuf.at[1]], gbuf, gsem).wait()
        for g in range(CH // LANES):
            vgroup = lax.bitcast_convert_type(
                ebuf[2, pl.ds(g * LANES, LANES)], jnp.float32)
            for j in range(LANES):
                e = g * LANES + j
                bvec = jnp.full((LANES,), vgroup[j], dtype=jnp.float32)
                for f in range(FV):
                    sl = pl.ds(f * LANES, LANES)
                    gbuf[e, sl] = gbuf[e, sl] * bvec
        pltpu.sync_copy(gbuf, acc.at[ebuf.at[0]], add=True)
        return carry

    lax.fori_loop(0, nk, chunk, 0)

    plsc.subcore_barrier()
    pltpu.sync_copy(acc.at[pl.ds(sid * ROWS_PER_TILE, ROWS_PER_TILE)],
                    out_hbm.at[cid, pl.ds(sid * ROWS_PER_TILE, ROWS_PER_TILE)])


_spmm = functools.partial(
    pl.kernel,
    out_type=jax.ShapeDtypeStruct((NC, N_PAD, NCLASS), jnp.float32),
    mesh=plsc.VectorSubcoreMesh(core_axis_name="c", subcore_axis_name="s",
                                num_cores=NC, num_subcores=NS),
    scratch_types=[
        pltpu.VMEM_SHARED((N_PAD, NCLASS), jnp.float32),
        pltpu.VMEM((3, CH), jnp.int32),
        pltpu.VMEM((CH, NCLASS), jnp.float32),
        pltpu.SemaphoreType.DMA,
    ],
)(_spmm_body)


def _combine_body(p_ref, o_ref):
    o_ref[...] = p_ref[0] + p_ref[1]


def _combine(partials):
    blk = 2000
    return pl.pallas_call(
        _combine_body,
        grid=(N // blk,),
        in_specs=[pl.BlockSpec((NC, blk, NCLASS), lambda i: (0, i, 0))],
        out_specs=pl.BlockSpec((blk, NCLASS), lambda i: (i, 0)),
        out_shape=jax.ShapeDtypeStruct((N, NCLASS), jnp.float32),
    )(partials)


def _skew(a):
    a = a.reshape(NS, 2 * NCHUNK, CH)
    c0 = jnp.pad(a[:, :K0], ((0, 0), (0, KMAX - K0), (0, 0)))
    c1 = jnp.pad(a[:, K0:], ((0, 0), (0, KMAX - K1), (0, 0)))
    return jnp.stack([c0, c1], axis=1).reshape(NW, KMAX, CH)


def kernel(x, S_indices, S_values, W, b):
    h = _linear(x, W, b)
    e_pad = NW * EPW - E
    rows = jnp.pad(S_indices[0], (0, e_pad))
    cols = jnp.pad(S_indices[1], (0, e_pad))
    vals = jnp.pad(S_values, (0, e_pad)).view(jnp.int32)
    epack = jnp.stack([_skew(rows), _skew(cols), _skew(vals)], axis=2)
    zeros = jnp.zeros((ROWS_PER_TILE, NCLASS), jnp.float32)
    partials = _spmm(h, epack, zeros)
    return _combine(partials)
